# Initial kernel scaffold; baseline (speedup 1.0000x reference)
import jax
import jax.numpy as jnp
from jax.experimental import pallas as pl


def kernel(input_ids, ImportanceFeatureMat, labels, embedding_table, VvT, phi_vs, bin_weight_difference, bin_weight_difference_start, w_word, w_sent):
    raise NotImplementedError("write your pallas kernel here")


# speedup vs baseline: 3952.7909x; 3952.7909x over previous
import functools

import numpy as np
import jax
import jax.numpy as jnp
from jax import lax
from jax.experimental import pallas as pl
from jax.experimental.pallas import tpu as pltpu
from jax.experimental.pallas import tpu_sc as plsc

_B, _S, _W, _D = 8, 16, 32, 64
_ND = _S * _W
_NV = 4096
_NODE = 32
_VT = 512
_NVT = _NV // _VT
_EDGES = [float(x) for x in np.linspace(-0.5, 0.99, 15)]


_NW = 32
_RPW = (_B * _ND) // _NW


def _sc_gather(table, idx):
  mesh = plsc.VectorSubcoreMesh(core_axis_name="c", subcore_axis_name="s")

  @functools.partial(
      pl.kernel,
      mesh=mesh,
      compiler_params=pltpu.CompilerParams(use_tc_tiling_on_sc=False),
      out_type=jax.ShapeDtypeStruct((_B * _ND, _D), jnp.float32),
      scratch_types=[
          pltpu.VMEM((_RPW,), jnp.int32),
          pltpu.VMEM((_RPW, _D), jnp.float32),
          pltpu.SemaphoreType.DMA,
      ],
  )
  def gather_kernel(table_hbm, idx_hbm, out_hbm, idx_v, rows_v, sem):
    wid = lax.axis_index("s") * 2 + lax.axis_index("c")
    base = wid * _RPW
    pltpu.sync_copy(idx_hbm.at[pl.ds(base, _RPW)], idx_v)
    pltpu.async_copy(table_hbm.at[idx_v], rows_v, sem).wait()
    pltpu.sync_copy(rows_v, out_hbm.at[pl.ds(base, _RPW)])

  return gather_kernel(table, idx)


def _tc_body(demb_ref, vvt_ref, imp_ref, phi_ref, ww_ref, ws_ref,
             diff_ref, start_ref, final_ref, attn_ref, attn_scr):
  vt = pl.program_id(1)

  @pl.when(vt == 0)
  def _():
    demb = demb_ref[...]
    wl = jnp.sum(demb * ww_ref[...], axis=1, keepdims=True)
    wl3 = wl.reshape(_S, _W, 1)
    wmax = jnp.max(wl3, axis=1, keepdims=True)
    we = jnp.exp(wl3 - wmax)
    wa = we / jnp.sum(we, axis=1, keepdims=True)
    sl = jnp.sum(imp_ref[...] * ws_ref[...], axis=1, keepdims=True)
    smax = jnp.max(sl, axis=0, keepdims=True)
    se = jnp.exp(sl - smax)
    sa = se / jnp.sum(se, axis=0, keepdims=True)
    attn = (wa * sa.reshape(_S, 1, 1)).reshape(_ND, 1)
    attn_scr[...] = attn
    attn_ref[...] = attn

  sim = jnp.dot(demb_ref[...], vvt_ref[...],
                preferred_element_type=jnp.float32)
  c0 = start_ref[0] + jnp.maximum(diff_ref[0], 0.0)
  w = jnp.full(sim.shape, 0.0, jnp.float32) + c0
  for k in range(1, 16):
    rd = jnp.maximum(diff_ref[k], 0.0)
    w = w + jnp.where(sim >= _EDGES[k - 1], rd, 0.0)
  weighted = jnp.sum(attn_scr[...] * w, axis=0, keepdims=True)
  part = lax.dot_general(weighted, phi_ref[...],
                         (((1,), (1,)), ((), ())),
                         preferred_element_type=jnp.float32)

  @pl.when(vt == 0)
  def _():
    final_ref[...] = part

  @pl.when(vt != 0)
  def _():
    final_ref[...] = final_ref[...] + part


def _tc_call(demb3, VvT, imp, phi, ww2, ws2, diff, start):
  return pl.pallas_call(
      _tc_body,
      grid=(_B, _NVT),
      in_specs=[
          pl.BlockSpec((None, _ND, _D), lambda b, v: (b, 0, 0)),
          pl.BlockSpec((_D, _VT), lambda b, v: (0, v)),
          pl.BlockSpec((None, _S, _D), lambda b, v: (b, 0, 0)),
          pl.BlockSpec((_NODE, _VT), lambda b, v: (0, v)),
          pl.BlockSpec((1, _D), lambda b, v: (0, 0)),
          pl.BlockSpec((1, _D), lambda b, v: (0, 0)),
          pl.BlockSpec(memory_space=pltpu.SMEM),
          pl.BlockSpec(memory_space=pltpu.SMEM),
      ],
      out_specs=[
          pl.BlockSpec((None, 1, _NODE), lambda b, v: (b, 0, 0)),
          pl.BlockSpec((None, _ND, 1), lambda b, v: (b, 0, 0)),
      ],
      out_shape=[
          jax.ShapeDtypeStruct((_B, 1, _NODE), jnp.float32),
          jax.ShapeDtypeStruct((_B, _ND, 1), jnp.float32),
      ],
      scratch_shapes=[pltpu.VMEM((_ND, 1), jnp.float32)],
  )(demb3, VvT, imp, phi, ww2, ws2, diff, start)


def kernel(input_ids, ImportanceFeatureMat, labels, embedding_table, VvT,
           phi_vs, bin_weight_difference, bin_weight_difference_start,
           w_word, w_sent):
  ids = input_ids.reshape(-1).astype(jnp.int32)
  demb = _sc_gather(embedding_table, ids)
  demb3 = demb.reshape(_B, _ND, _D)
  final, attn = _tc_call(
      demb3, VvT, ImportanceFeatureMat, phi_vs,
      w_word.reshape(1, _D), w_sent.reshape(1, _D),
      bin_weight_difference, bin_weight_difference_start)
  return final.reshape(_B, _NODE), attn.reshape(_B, _ND)


# speedup vs baseline: 4563.5203x; 1.1545x over previous
import functools

import numpy as np
import jax
import jax.numpy as jnp
from jax import lax
from jax.experimental import pallas as pl
from jax.experimental.pallas import tpu as pltpu
from jax.experimental.pallas import tpu_sc as plsc

_B, _S, _W, _D = 8, 16, 32, 64
_ND = _S * _W
_NV = 4096
_NODE = 32
_VT = 1024
_NVT = _NV // _VT
_EDGES = [float(x) for x in np.linspace(-0.5, 0.99, 15)]


_NW = 32
_RPW = (_B * _ND) // _NW


def _sc_gather(table, idx):
  mesh = plsc.VectorSubcoreMesh(core_axis_name="c", subcore_axis_name="s")

  @functools.partial(
      pl.kernel,
      mesh=mesh,
      compiler_params=pltpu.CompilerParams(use_tc_tiling_on_sc=False),
      out_type=jax.ShapeDtypeStruct((_B * _ND, _D), jnp.float32),
      scratch_types=[
          pltpu.VMEM((_RPW,), jnp.int32),
          pltpu.VMEM((_RPW, _D), jnp.float32),
          pltpu.SemaphoreType.DMA,
      ],
  )
  def gather_kernel(table_hbm, idx_hbm, out_hbm, idx_v, rows_v, sem):
    wid = lax.axis_index("s") * 2 + lax.axis_index("c")
    base = wid * _RPW
    pltpu.sync_copy(idx_hbm.at[pl.ds(base, _RPW)], idx_v)
    pltpu.async_copy(table_hbm.at[idx_v], rows_v, sem).wait()
    pltpu.sync_copy(rows_v, out_hbm.at[pl.ds(base, _RPW)])

  return gather_kernel(table, idx)


def _tc_body(demb_ref, vvt_ref, imp_ref, phi_ref, ww_ref, ws_ref,
             diff_ref, start_ref, final_ref, attn_ref, attn_scr):
  vt = pl.program_id(1)

  @pl.when(vt == 0)
  def _():
    demb = demb_ref[...]
    wl = jnp.sum(demb * ww_ref[...], axis=1, keepdims=True)
    wl3 = wl.reshape(_S, _W, 1)
    wmax = jnp.max(wl3, axis=1, keepdims=True)
    we = jnp.exp(wl3 - wmax)
    wa = we / jnp.sum(we, axis=1, keepdims=True)
    sl = jnp.sum(imp_ref[...] * ws_ref[...], axis=1, keepdims=True)
    smax = jnp.max(sl, axis=0, keepdims=True)
    se = jnp.exp(sl - smax)
    sa = se / jnp.sum(se, axis=0, keepdims=True)
    attn = (wa * sa.reshape(_S, 1, 1)).reshape(_ND, 1)
    attn_scr[...] = attn
    attn_ref[...] = attn

  sim = jnp.dot(demb_ref[...], vvt_ref[...],
                preferred_element_type=jnp.float32)
  bw = [start_ref[0] + jnp.maximum(diff_ref[0], 0.0)]
  for k in range(1, 16):
    bw.append(bw[-1] + jnp.maximum(diff_ref[k], 0.0))
  b3 = sim >= _EDGES[7]
  b2 = sim >= jnp.where(b3, _EDGES[11], _EDGES[3])
  e1 = jnp.where(b3, jnp.where(b2, _EDGES[13], _EDGES[9]),
                 jnp.where(b2, _EDGES[5], _EDGES[1]))
  b1 = sim >= e1
  e0hi = jnp.where(b3, jnp.where(b2, _EDGES[14], _EDGES[10]),
                   jnp.where(b2, _EDGES[6], _EDGES[2]))
  e0lo = jnp.where(b3, jnp.where(b2, _EDGES[12], _EDGES[8]),
                   jnp.where(b2, _EDGES[4], _EDGES[0]))
  b0 = sim >= jnp.where(b1, e0hi, e0lo)
  t = [jnp.where(b0, bw[2 * i + 1], bw[2 * i]) for i in range(8)]
  u = [jnp.where(b1, t[2 * j + 1], t[2 * j]) for j in range(4)]
  p = [jnp.where(b2, u[1], u[0]), jnp.where(b2, u[3], u[2])]
  w = jnp.where(b3, p[1], p[0])
  weighted = lax.dot_general(attn_scr[...], w,
                             (((0,), (0,)), ((), ())),
                             preferred_element_type=jnp.float32)
  part = lax.dot_general(weighted, phi_ref[...],
                         (((1,), (1,)), ((), ())),
                         preferred_element_type=jnp.float32)

  @pl.when(vt == 0)
  def _():
    final_ref[...] = part

  @pl.when(vt != 0)
  def _():
    final_ref[...] = final_ref[...] + part


def _tc_call(demb3, VvT, imp, phi, ww2, ws2, diff, start):
  return pl.pallas_call(
      _tc_body,
      grid=(_B, _NVT),
      in_specs=[
          pl.BlockSpec((_ND, _D), lambda b, v: (b, 0)),
          pl.BlockSpec((_D, _VT), lambda b, v: (0, v)),
          pl.BlockSpec((None, _S, _D), lambda b, v: (b, 0, 0)),
          pl.BlockSpec((_NODE, _VT), lambda b, v: (0, v)),
          pl.BlockSpec((1, _D), lambda b, v: (0, 0)),
          pl.BlockSpec((1, _D), lambda b, v: (0, 0)),
          pl.BlockSpec(memory_space=pltpu.SMEM),
          pl.BlockSpec(memory_space=pltpu.SMEM),
      ],
      out_specs=[
          pl.BlockSpec((None, 1, _NODE), lambda b, v: (b, 0, 0)),
          pl.BlockSpec((None, _ND, 1), lambda b, v: (b, 0, 0)),
      ],
      out_shape=[
          jax.ShapeDtypeStruct((_B, 1, _NODE), jnp.float32),
          jax.ShapeDtypeStruct((_B, _ND, 1), jnp.float32),
      ],
      scratch_shapes=[pltpu.VMEM((_ND, 1), jnp.float32)],
  )(demb3, VvT, imp, phi, ww2, ws2, diff, start)


def kernel(input_ids, ImportanceFeatureMat, labels, embedding_table, VvT,
           phi_vs, bin_weight_difference, bin_weight_difference_start,
           w_word, w_sent):
  ids = input_ids.reshape(-1).astype(jnp.int32)
  demb = _sc_gather(embedding_table, ids)
  final, attn = _tc_call(
      demb, VvT, ImportanceFeatureMat, phi_vs,
      w_word.reshape(1, _D), w_sent.reshape(1, _D),
      bin_weight_difference, bin_weight_difference_start)
  return final.reshape(_B, _NODE), attn.reshape(_B, _ND)
